# Initial kernel scaffold; baseline (speedup 1.0000x reference)
import jax
import jax.numpy as jnp
from jax.experimental import pallas as pl


def kernel(theta, t, Y_train):
    raise NotImplementedError("write your pallas kernel here")


# speedup vs baseline: 6.2558x; 6.2558x over previous
import functools

import jax
import jax.numpy as jnp
from jax.experimental import pallas as pl
from jax.experimental.pallas import tpu as pltpu

_EPS = 0.5
_N = 256
_ROWS = 1024


def _lane_cumsum(x):
    r, n = x.shape
    cs = x
    k = 1
    while k < n:
        shifted = jnp.concatenate(
            [jnp.zeros((r, k), x.dtype), cs[:, : n - k]], axis=1)
        cs = cs + shifted
        k *= 2
    return cs


def _body(theta_ref, t_ref, y_ref, out_ref):
    th = theta_ref[...]
    t = t_ref[...]
    cs = _lane_cumsum(th)
    m = jnp.sum((cs < t).astype(jnp.int32), axis=1, keepdims=True)
    idx = jnp.minimum(m, _N - 1)
    lane = jax.lax.broadcasted_iota(jnp.int32, (1, _N), 1)
    lt = (lane < idx).astype(th.dtype)
    eq = (lane == idx).astype(th.dtype)
    cumsum_j = jnp.sum(th * lt, axis=1, keepdims=True)
    w = jnp.sum(th * eq, axis=1, keepdims=True)
    yj = jnp.sum(y_ref[...] * eq, axis=1, keepdims=True)
    s = (t - cumsum_j) / w
    out_ref[...] = yj - _EPS + (2.0 * _EPS) * s


@functools.partial(jax.jit, static_argnames=())
def kernel(theta, t, Y_train):
    batch, n = theta.shape
    assert n == _N
    t2 = t.reshape(batch, 1)
    y2 = Y_train.reshape(1, _N)
    grid = (batch // _ROWS,)
    out = pl.pallas_call(
        _body,
        grid=grid,
        in_specs=[
            pl.BlockSpec((_ROWS, _N), lambda i: (i, 0)),
            pl.BlockSpec((_ROWS, 1), lambda i: (i, 0)),
            pl.BlockSpec((1, _N), lambda i: (0, 0)),
        ],
        out_specs=pl.BlockSpec((_ROWS, 1), lambda i: (i, 0)),
        out_shape=jax.ShapeDtypeStruct((batch, 1), theta.dtype),
        compiler_params=pltpu.CompilerParams(
            dimension_semantics=("arbitrary",),
        ),
    )(theta, t2, y2)
    return out.reshape(batch)
